# baseline (speedup 1.0000x reference)
import functools

import jax
import jax.numpy as jnp
from jax import lax
from jax.experimental import pallas as pl
from jax.experimental.pallas import tpu as pltpu
from jax.experimental.pallas import tpu_sc as plsc

_NC = 2
_NS = 16
_NW = _NC * _NS
_L = 16
_CHUNK = 128


def _make_sc_kernel(B, K, dtype):
    b_per_w = B // _NW
    n_chunks = b_per_w // _CHUNK

    mesh = plsc.VectorSubcoreMesh(core_axis_name="c", subcore_axis_name="s")

    @functools.partial(
        pl.kernel,
        mesh=mesh,
        out_type=jax.ShapeDtypeStruct((_NW, b_per_w), dtype),
        scratch_types=[
            pltpu.VMEM((n_chunks, _CHUNK), jnp.int32),
            pltpu.VMEM((n_chunks, _CHUNK), jnp.int32),
            pltpu.VMEM((b_per_w, K), dtype),
            pltpu.VMEM((b_per_w, K), dtype),
            pltpu.VMEM((b_per_w,), dtype),
            pltpu.SemaphoreType.DMA,
        ],
        compiler_params=pltpu.CompilerParams(
            use_tc_tiling_on_sc=False, needs_layout_passes=False),
    )
    def sc_kernel(uid_hbm, iid_hbm, p_hbm, q_hbm, out_hbm,
                  uidx_v, iidx_v, p_v, q_v, out_v, sem):
        wid = lax.axis_index("s") * _NC + lax.axis_index("c")

        pltpu.sync_copy(uid_hbm.at[wid], uidx_v)
        pltpu.sync_copy(iid_hbm.at[wid], iidx_v)

        copies = []
        for j in range(n_chunks):
            copies.append(pltpu.async_copy(
                p_hbm.at[uidx_v.at[j]], p_v.at[pl.ds(j * _CHUNK, _CHUNK)], sem))
            copies.append(pltpu.async_copy(
                q_hbm.at[iidx_v.at[j]], q_v.at[pl.ds(j * _CHUNK, _CHUNK)], sem))
        for cp in copies:
            cp.wait()

        lanes = lax.iota(jnp.int32, 16)

        def group_body(g, carry):
            rows = g * _L + lanes
            acc = jnp.zeros((16,), dtype)
            for k in range(K):
                kv = jnp.full((16,), k, jnp.int32)
                pv = plsc.load_gather(p_v, [rows, kv])
                qv = plsc.load_gather(q_v, [rows, kv])
                acc = acc + pv * qv
            plsc.store_scatter(out_v, [rows], acc)
            return carry

        lax.fori_loop(0, b_per_w // _L, group_body, 0)
        pltpu.sync_copy(out_v, out_hbm.at[wid])

    return sc_kernel


def kernel(user_ids, item_ids, P, Q):
    B = user_ids.shape[0]
    K = P.shape[1]
    b_per_w = B // _NW
    uid = user_ids.astype(jnp.int32).reshape(_NW, b_per_w // _CHUNK, _CHUNK)
    iid = item_ids.astype(jnp.int32).reshape(_NW, b_per_w // _CHUNK, _CHUNK)
    out = _make_sc_kernel(B, K, P.dtype)(uid, iid, P, Q)
    return out.reshape(B)
